# baseline (speedup 1.0000x reference)
import functools

import jax
import jax.numpy as jnp
from jax import lax
from jax.experimental import pallas as pl
from jax.experimental.pallas import tpu as pltpu
from jax.experimental.pallas import tpu_sc as plsc

B, HW, DIM = 64, 1024, 64
K = 9 * 114
KPAD = 1152
T = B * HW

NCH = 4
TCH = T // NCH
TB = 4096
NB = TCH // TB

NW = 32
RPW = TCH // NW
CH = 128
NSTREAM = RPW // CH


def _idx_body(zt_ref, a_ref, o_ref):
    zt = zt_ref[...]
    nrm = jnp.sqrt(jnp.sum(zt * zt, axis=0, keepdims=True))
    zn = zt / jnp.maximum(nrm, 1e-12)
    simsT = jnp.dot(a_ref[...], zn.astype(jnp.bfloat16),
                    preferred_element_type=jnp.float32)
    o_ref[0, 0] = (KPAD - 1) - jnp.argmax(simsT, axis=0).astype(jnp.int32)


def _gather_body(a_hbm, idx_hbm, out_hbm, idx_v, rows_v, sem0, sem1):
    wid = lax.axis_index("s") * 2 + lax.axis_index("c")
    base = wid * RPW
    pltpu.sync_copy(idx_hbm.at[pl.ds(base, RPW)], idx_v)
    sems = (sem0, sem1)
    cps = [None, None]
    cps[0] = pltpu.async_copy(a_hbm.at[idx_v.at[pl.ds(0, CH)]],
                              rows_v.at[0], sems[0])
    for j in range(NSTREAM):
        nxt = (j + 1) % 2
        if j + 1 < NSTREAM:
            cps[nxt] = pltpu.async_copy(
                a_hbm.at[idx_v.at[pl.ds((j + 1) * CH, CH)]],
                rows_v.at[nxt], sems[nxt])
        cps[j % 2].wait()
        pltpu.sync_copy(rows_v.at[j % 2],
                        out_hbm.at[pl.ds(base + j * CH, CH)])


_sc_gather = functools.partial(
    pl.kernel,
    mesh=plsc.VectorSubcoreMesh(core_axis_name="c", subcore_axis_name="s"),
    out_type=jax.ShapeDtypeStruct((TCH, 128), jnp.float32),
    scratch_types=[
        pltpu.VMEM((RPW,), jnp.int32),
        pltpu.VMEM((2, CH, 128), jnp.float32),
        pltpu.SemaphoreType.DMA,
        pltpu.SemaphoreType.DMA,
    ],
)(_gather_body)


def kernel(z, attractors):
    A = attractors.reshape(-1, DIM)
    a_pad = jnp.concatenate(
        [A, jnp.broadcast_to(A[:1], (KPAD - K, DIM))], axis=0)
    a_rev = a_pad[::-1].astype(jnp.bfloat16)
    a_wide = jnp.pad(a_pad, ((0, 0), (0, 128 - DIM)))
    zf = z.reshape(T, DIM)
    outs = []
    for c in range(NCH):
        zt_c = zf[c * TCH:(c + 1) * TCH].T
        idx_c = pl.pallas_call(
            _idx_body,
            grid=(NB,),
            in_specs=[
                pl.BlockSpec((DIM, TB), lambda i: (0, i)),
                pl.BlockSpec((KPAD, DIM), lambda i: (0, 0)),
            ],
            out_specs=pl.BlockSpec((1, 1, TB), lambda i: (i, 0, 0)),
            out_shape=jax.ShapeDtypeStruct((NB, 1, TB), jnp.int32),
        )(zt_c, a_rev)
        outs.append(_sc_gather(a_wide, idx_c.reshape(TCH)))
    out = jnp.concatenate(outs, axis=0)
    return out[:, :DIM].reshape(B, HW, DIM)


# speedup vs baseline: 1.2126x; 1.2126x over previous
import functools

import jax
import jax.numpy as jnp
from jax import lax
from jax.experimental import pallas as pl
from jax.experimental.pallas import tpu as pltpu
from jax.experimental.pallas import tpu_sc as plsc

B, HW, DIM = 64, 1024, 64
K = 9 * 114
KPAD = 1152
T = B * HW

NCH = 4
TCH = T // NCH
TB = 4096
NB = TCH // TB

NW = 32
RPW = TCH // NW
CH = 128
NSTREAM = RPW // CH


def _idx_body(zt_ref, a_ref, o_ref):
    zt = zt_ref[...]
    nrm = jnp.sqrt(jnp.sum(zt * zt, axis=0, keepdims=True))
    zn = zt / jnp.maximum(nrm, 1e-12)
    simsT = jnp.dot(a_ref[...], zn.astype(jnp.bfloat16),
                    preferred_element_type=jnp.float32)
    o_ref[0, 0] = (KPAD - 1) - jnp.argmax(simsT, axis=0).astype(jnp.int32)


def _gather_body(a_hbm, idx_hbm, out_hbm, idx_v, rows_v, sem0, sem1):
    wid = lax.axis_index("s") * 2 + lax.axis_index("c")
    base = wid * RPW
    pltpu.sync_copy(idx_hbm.at[pl.ds(base, RPW)], idx_v)
    sems = (sem0, sem1)
    cps = [None, None]
    cps[0] = pltpu.async_copy(a_hbm.at[idx_v.at[pl.ds(0, CH)]],
                              rows_v.at[0], sems[0])
    for j in range(NSTREAM):
        nxt = (j + 1) % 2
        if j + 1 < NSTREAM:
            cps[nxt] = pltpu.async_copy(
                a_hbm.at[idx_v.at[pl.ds((j + 1) * CH, CH)]],
                rows_v.at[nxt], sems[nxt])
        cps[j % 2].wait()
        pltpu.sync_copy(rows_v.at[j % 2],
                        out_hbm.at[pl.ds(base + j * CH, CH)])


_sc_gather = functools.partial(
    pl.kernel,
    mesh=plsc.VectorSubcoreMesh(core_axis_name="c", subcore_axis_name="s"),
    out_type=jax.ShapeDtypeStruct((TCH, 128), jnp.float32),
    scratch_types=[
        pltpu.VMEM((RPW,), jnp.int32),
        pltpu.VMEM((2, CH, 128), jnp.float32),
        pltpu.SemaphoreType.DMA,
        pltpu.SemaphoreType.DMA,
    ],
)(_gather_body)


def kernel(z, attractors):
    A = attractors.reshape(-1, DIM)
    a_pad = jnp.concatenate(
        [A, jnp.broadcast_to(A[:1], (KPAD - K, DIM))], axis=0)
    a_rev = a_pad[::-1].astype(jnp.bfloat16)
    a_wide = jnp.pad(a_pad, ((0, 0), (0, 128 - DIM)))
    zt = z.reshape(T, DIM).T
    outs = []
    for c in range(NCH):
        idx_c = pl.pallas_call(
            _idx_body,
            grid=(NB,),
            in_specs=[
                pl.BlockSpec((DIM, TB), lambda i, c=c: (0, c * NB + i)),
                pl.BlockSpec((KPAD, DIM), lambda i: (0, 0)),
            ],
            out_specs=pl.BlockSpec((1, 1, TB), lambda i: (i, 0, 0)),
            out_shape=jax.ShapeDtypeStruct((NB, 1, TB), jnp.int32),
        )(zt, a_rev)
        out_c = _sc_gather(a_wide, idx_c.reshape(TCH))
        outs.append(out_c[:, :DIM].reshape(TCH // HW, HW, DIM))
    return jnp.concatenate(outs, axis=0)
